# baseline (speedup 1.0000x reference)
import functools

import jax
import jax.numpy as jnp
from jax import lax
from jax.experimental import pallas as pl
from jax.experimental.pallas import tpu as pltpu
from jax.experimental.pallas import tpu_sc as plsc

_ROWS = 16384
_COLS = 4096
_NMASK = 64
_NW = 32
_RPW = _ROWS // _NW
_C = 8
_G = _RPW // _C
_NBUF = 2


def _sc_body(x_hbm, mask_hbm, out_hbm, mask_v, *rest):
    bufs = rest[:_NBUF]
    rsems = rest[_NBUF : 2 * _NBUF]
    wsems = rest[2 * _NBUF :]
    nc = lax.axis_size("c")
    wid = lax.axis_index("s") * nc + lax.axis_index("c")
    base = wid * _RPW
    pltpu.sync_copy(mask_hbm, mask_v)
    zeros16 = jnp.zeros((16,), jnp.float32)

    def start_read(g, b):
        pltpu.make_async_copy(
            x_hbm.at[pl.ds(base + g * _C, _C)], bufs[b], rsems[b]
        ).start()

    def wait_read(g, b):
        pltpu.make_async_copy(
            x_hbm.at[pl.ds(base + g * _C, _C)], bufs[b], rsems[b]
        ).wait()

    def start_write(g, b):
        pltpu.make_async_copy(
            bufs[b], out_hbm.at[pl.ds(base + g * _C, _C)], wsems[b]
        ).start()

    def wait_write(g, b):
        pltpu.make_async_copy(
            bufs[b], out_hbm.at[pl.ds(base + g * _C, _C)], wsems[b]
        ).wait()

    def zero_cols(b):
        for r in range(_C):
            rvec = jnp.full((16,), r, jnp.int32)
            for k in range(_NMASK // 16):
                idxs = mask_v[pl.ds(k * 16, 16)]
                plsc.store_scatter(bufs[b], [rvec, idxs], zeros16)

    def step(g, b, wait_prev_write, read_ahead):
        wait_read(g, b)
        zero_cols(b)
        start_write(g, b)
        if wait_prev_write:
            wait_write(g - 1, (b - 1) % _NBUF)
        if read_ahead:
            start_read(g + _NBUF - 1, (b - 1) % _NBUF)

    for g in range(_NBUF - 1):
        start_read(g, g)
    step(0, 0, wait_prev_write=False, read_ahead=True)
    for b in range(1, _NBUF):
        step(b, b, wait_prev_write=True, read_ahead=True)

    def block(gg, carry):
        for b in range(_NBUF):
            g = gg * _NBUF + b
            step(g, b, wait_prev_write=True, read_ahead=True)
        return carry

    lax.fori_loop(1, _G // _NBUF - 1, block, 0)

    for b in range(_NBUF):
        g = _G - _NBUF + b
        step(g, b, wait_prev_write=True, read_ahead=(g + _NBUF - 1 < _G))
    wait_write(_G - 1, (_G - 1) % _NBUF)


@functools.cache
def _make_sc_kernel():
    return pl.kernel(
        _sc_body,
        out_type=jax.ShapeDtypeStruct((_ROWS, _COLS), jnp.float32),
        mesh=plsc.VectorSubcoreMesh(core_axis_name="c", subcore_axis_name="s"),
        scratch_types=(
            [pltpu.VMEM((_NMASK,), jnp.int32)]
            + [pltpu.VMEM((_C, _COLS), jnp.float32) for _ in range(_NBUF)]
            + [pltpu.SemaphoreType.DMA for _ in range(2 * _NBUF)]
        ),
        compiler_params=pltpu.CompilerParams(needs_layout_passes=False),
    )


def kernel(x, mask):
    return _make_sc_kernel()(x, mask)


# speedup vs baseline: 1.0583x; 1.0583x over previous
import functools

import jax
import jax.numpy as jnp
from jax import lax
from jax.experimental import pallas as pl
from jax.experimental.pallas import tpu as pltpu
from jax.experimental.pallas import tpu_sc as plsc

_ROWS = 16384
_COLS = 4096
_NMASK = 64
_NW = 32
_RPW = _ROWS // _NW
_C = 2
_G = _RPW // _C
_NBUF = 8


def _sc_body(x_hbm, mask_hbm, out_hbm, mask_v, *rest):
    bufs = rest[:_NBUF]
    rsems = rest[_NBUF : 2 * _NBUF]
    wsems = rest[2 * _NBUF :]
    nc = lax.axis_size("c")
    wid = lax.axis_index("s") * nc + lax.axis_index("c")
    base = wid * _RPW
    pltpu.sync_copy(mask_hbm, mask_v)
    zeros16 = jnp.zeros((16,), jnp.float32)

    def start_read(g, b):
        pltpu.make_async_copy(
            x_hbm.at[pl.ds(base + g * _C, _C)], bufs[b], rsems[b]
        ).start()

    def wait_read(g, b):
        pltpu.make_async_copy(
            x_hbm.at[pl.ds(base + g * _C, _C)], bufs[b], rsems[b]
        ).wait()

    def start_write(g, b):
        pltpu.make_async_copy(
            bufs[b], out_hbm.at[pl.ds(base + g * _C, _C)], wsems[b]
        ).start()

    def wait_write(g, b):
        pltpu.make_async_copy(
            bufs[b], out_hbm.at[pl.ds(base + g * _C, _C)], wsems[b]
        ).wait()

    def zero_cols(b):
        for r in range(_C):
            rvec = jnp.full((16,), r, jnp.int32)
            for k in range(_NMASK // 16):
                idxs = mask_v[pl.ds(k * 16, 16)]
                plsc.store_scatter(bufs[b], [rvec, idxs], zeros16)

    def step(g, b, wait_prev_write, read_ahead):
        wait_read(g, b)
        zero_cols(b)
        start_write(g, b)
        if wait_prev_write:
            wait_write(g - 1, (b - 1) % _NBUF)
        if read_ahead:
            start_read(g + _NBUF - 1, (b - 1) % _NBUF)

    for g in range(_NBUF - 1):
        start_read(g, g)
    step(0, 0, wait_prev_write=False, read_ahead=True)
    for b in range(1, _NBUF):
        step(b, b, wait_prev_write=True, read_ahead=True)

    def block(gg, carry):
        for b in range(_NBUF):
            g = gg * _NBUF + b
            step(g, b, wait_prev_write=True, read_ahead=True)
        return carry

    lax.fori_loop(1, _G // _NBUF - 1, block, 0)

    for b in range(_NBUF):
        g = _G - _NBUF + b
        step(g, b, wait_prev_write=True, read_ahead=(g + _NBUF - 1 < _G))
    wait_write(_G - 1, (_G - 1) % _NBUF)


@functools.cache
def _make_sc_kernel():
    return pl.kernel(
        _sc_body,
        out_type=jax.ShapeDtypeStruct((_ROWS, _COLS), jnp.float32),
        mesh=plsc.VectorSubcoreMesh(core_axis_name="c", subcore_axis_name="s"),
        scratch_types=(
            [pltpu.VMEM((_NMASK,), jnp.int32)]
            + [pltpu.VMEM((_C, _COLS), jnp.float32) for _ in range(_NBUF)]
            + [pltpu.SemaphoreType.DMA for _ in range(2 * _NBUF)]
        ),
        compiler_params=pltpu.CompilerParams(needs_layout_passes=False),
    )


def kernel(x, mask):
    return _make_sc_kernel()(x, mask)
